# baseline (speedup 1.0000x reference)
import functools

import jax
import jax.numpy as jnp
from jax import lax
from jax.experimental import pallas as pl
from jax.experimental.pallas import tpu as pltpu
from jax.experimental.pallas import tpu_sc as plsc

N = 10000
C = 128
NG = 64
NW = 32
CHUNK = 104
NCH = 3
KPW = CHUNK * NCH
REM = N - NW * KPW
GPAD = 80


def _sc_body(emb_hbm, gidx_hbm, batch_hbm, out_hbm,
             gidx_v, bidx_v, rows_v, bidx_x, rows_x, zbuf_v, acc_sh,
             sem_sc, sem_x, *gsems):
    cid = lax.axis_index("c")
    sid = lax.axis_index("s")
    wid = sid * 2 + cid
    base = wid * KPW

    for r in range(GPAD // 16):
        for k in range(C // 16):
            zbuf_v[r, pl.ds(k * 16, 16)] = jnp.zeros((16,), jnp.float32)
    pltpu.sync_copy(zbuf_v, acc_sh.at[pl.ds(sid * (GPAD // 16), GPAD // 16)])

    pltpu.sync_copy(gidx_hbm.at[pl.ds(base, KPW)], gidx_v)
    for j in range(NCH):
        pltpu.sync_copy(batch_hbm.at[pl.ds(base + j * CHUNK, CHUNK)],
                        bidx_v.at[j])
    gathers = [
        pltpu.async_copy(emb_hbm.at[gidx_v.at[pl.ds(j * CHUNK, CHUNK)]],
                         rows_v.at[j], gsems[j])
        for j in range(NCH)
    ]
    plsc.subcore_barrier()

    scatters = []
    for j in range(NCH):
        gathers[j].wait()
        scatters.append(pltpu.async_copy(
            rows_v.at[j], acc_sh.at[bidx_v.at[j]], sem_sc, add=True))
    for d in scatters:
        d.wait()

    @pl.when(wid == NW - 1)
    def _tail():
        tb = NW * KPW
        pltpu.sync_copy(gidx_hbm.at[pl.ds(tb, REM)],
                        gidx_v.at[pl.ds(0, REM)])
        pltpu.sync_copy(batch_hbm.at[pl.ds(tb, REM)], bidx_x.at[0])
        pltpu.async_copy(emb_hbm.at[gidx_v.at[pl.ds(0, REM)]], rows_x,
                         sem_x).wait()
        pltpu.sync_copy(rows_x, acc_sh.at[bidx_x.at[0]], add=True)

    plsc.subcore_barrier()

    @pl.when(sid == 0)
    def _writeback():
        pltpu.sync_copy(acc_sh.at[pl.ds(0, NG)], out_hbm.at[cid])


@functools.cache
def _sc_segsum():
    mesh = plsc.VectorSubcoreMesh(core_axis_name="c", subcore_axis_name="s")
    return pl.kernel(
        _sc_body,
        out_type=jax.ShapeDtypeStruct((2, NG, C), jnp.float32),
        mesh=mesh,
        scratch_types=[
            pltpu.VMEM((KPW,), jnp.int32),
            pltpu.VMEM((NCH, CHUNK), jnp.int32),
            pltpu.VMEM((NCH, CHUNK, C), jnp.float32),
            pltpu.VMEM((1, REM), jnp.int32),
            pltpu.VMEM((REM, C), jnp.float32),
            pltpu.VMEM((GPAD // 16, C), jnp.float32),
            pltpu.VMEM_SHARED((GPAD, C), jnp.float32),
            pltpu.SemaphoreType.DMA,
            pltpu.SemaphoreType.DMA,
        ] + [pltpu.SemaphoreType.DMA] * NCH,
    )


def _tc_acts(batch_ref, actsT_ref, peWT_ref, peb_ref, out_ref):
    onehot = (batch_ref[...] == lax.broadcasted_iota(
        jnp.int32, (NG, N), 0)).astype(jnp.float32)
    segacts = lax.dot_general(onehot, actsT_ref[...],
                              (((1,), (1,)), ((), ())))
    counts = jnp.sum(onehot, axis=1, keepdims=True)
    out_ref[...] = lax.dot_general(segacts, peWT_ref[...],
                                   (((1,), (0,)), ((), ()))
                                   ) + counts * peb_ref[...]


def _tc_mlp(part_ref, act_ref, W1_ref, b1_ref, W2_ref, b2T_ref, out_ref):
    pooled = part_ref[0] + part_ref[1] + act_ref[...]
    h1 = lax.dot_general(pooled, W1_ref[...],
                         (((1,), (1,)), ((), ()))) + b1_ref[...]
    h1 = jnp.maximum(h1, 0.0)
    yT = lax.dot_general(W2_ref[...], h1,
                         (((1,), (1,)), ((), ()))) + b2T_ref[...]
    m = jnp.max(yT, axis=0, keepdims=True)
    lse = m + jnp.log(jnp.sum(jnp.exp(yT - m), axis=0, keepdims=True))
    out_ref[...] = yT - lse


def kernel(params, acts, sign, global_idx, edge_index, batch):
    p = params
    gidx32 = global_idx.astype(jnp.int32)
    batch32 = batch.astype(jnp.int32)

    act_part = pl.pallas_call(
        _tc_acts,
        out_shape=jax.ShapeDtypeStruct((NG, C), jnp.float32),
    )(batch32.reshape(1, N), acts.T, p['pe_W'].T, p['pe_b'].reshape(1, C))

    partials = _sc_segsum()(p['emb'], gidx32, batch32)

    outT = pl.pallas_call(
        _tc_mlp,
        out_shape=jax.ShapeDtypeStruct((2, NG), jnp.float32),
    )(partials, act_part, p['fc1_W'], p['fc1_b'].reshape(1, 2 * C),
      p['fc2_W'], p['fc2_b'].reshape(2, 1))
    return outT.T


# speedup vs baseline: 1.0457x; 1.0457x over previous
import functools

import jax
import jax.numpy as jnp
from jax import lax
from jax.experimental import pallas as pl
from jax.experimental.pallas import tpu as pltpu
from jax.experimental.pallas import tpu_sc as plsc

N = 10000
C = 128
NG = 64
NW = 32
CHUNK = 104
NCH = 3
KPW = CHUNK * NCH
REM = N - NW * KPW
GPAD = 80


def _sc_body(emb_hbm, gidx_hbm, batch_hbm, out_hbm,
             gidx_v, bidx_v, rows_v, bidx_x, rows_x, zbuf_v, acc_sh,
             sem_sc, sem_x, sem_gi, sem_bi, *gsems):
    cid = lax.axis_index("c")
    sid = lax.axis_index("s")
    wid = sid * 2 + cid
    base = wid * KPW

    gload = pltpu.async_copy(gidx_hbm.at[pl.ds(base, KPW)], gidx_v, sem_gi)
    bload = pltpu.async_copy(batch_hbm.at[pl.ds(base, KPW)], bidx_v, sem_bi)
    for r in range(GPAD // 16):
        for k in range(C // 16):
            zbuf_v[r, pl.ds(k * 16, 16)] = jnp.zeros((16,), jnp.float32)
    pltpu.sync_copy(zbuf_v, acc_sh.at[pl.ds(sid * (GPAD // 16), GPAD // 16)])

    gload.wait()
    gathers = [
        pltpu.async_copy(emb_hbm.at[gidx_v.at[pl.ds(j * CHUNK, CHUNK)]],
                         rows_v.at[j], gsems[j])
        for j in range(NCH)
    ]
    bload.wait()
    plsc.subcore_barrier()

    scatters = []
    for j in range(NCH):
        gathers[j].wait()
        scatters.append(pltpu.async_copy(
            rows_v.at[j], acc_sh.at[bidx_v.at[pl.ds(j * CHUNK, CHUNK)]],
            sem_sc, add=True))
    for d in scatters:
        d.wait()

    @pl.when(wid == NW - 1)
    def _tail():
        tb = NW * KPW
        pltpu.sync_copy(gidx_hbm.at[pl.ds(tb, REM)],
                        gidx_v.at[pl.ds(0, REM)])
        pltpu.sync_copy(batch_hbm.at[pl.ds(tb, REM)], bidx_x.at[0])
        pltpu.async_copy(emb_hbm.at[gidx_v.at[pl.ds(0, REM)]], rows_x,
                         sem_x).wait()
        pltpu.sync_copy(rows_x, acc_sh.at[bidx_x.at[0]], add=True)

    plsc.subcore_barrier()

    @pl.when(sid == 0)
    def _writeback():
        pltpu.sync_copy(acc_sh.at[pl.ds(0, NG)], out_hbm.at[cid])


@functools.cache
def _sc_segsum():
    mesh = plsc.VectorSubcoreMesh(core_axis_name="c", subcore_axis_name="s")
    return pl.kernel(
        _sc_body,
        out_type=jax.ShapeDtypeStruct((2, NG, C), jnp.float32),
        mesh=mesh,
        scratch_types=[
            pltpu.VMEM((KPW,), jnp.int32),
            pltpu.VMEM((KPW,), jnp.int32),
            pltpu.VMEM((NCH, CHUNK, C), jnp.float32),
            pltpu.VMEM((1, REM), jnp.int32),
            pltpu.VMEM((REM, C), jnp.float32),
            pltpu.VMEM((GPAD // 16, C), jnp.float32),
            pltpu.VMEM_SHARED((GPAD, C), jnp.float32),
            pltpu.SemaphoreType.DMA,
            pltpu.SemaphoreType.DMA,
            pltpu.SemaphoreType.DMA,
            pltpu.SemaphoreType.DMA,
        ] + [pltpu.SemaphoreType.DMA] * NCH,
    )


def _tc_acts(batch_ref, actsT_ref, peWT_ref, peb_ref, out_ref):
    onehot = (batch_ref[...] == lax.broadcasted_iota(
        jnp.int32, (NG, N), 0)).astype(jnp.float32)
    segacts = lax.dot_general(onehot, actsT_ref[...],
                              (((1,), (1,)), ((), ())))
    counts = jnp.sum(onehot, axis=1, keepdims=True)
    out_ref[...] = lax.dot_general(segacts, peWT_ref[...],
                                   (((1,), (0,)), ((), ()))
                                   ) + counts * peb_ref[...]


def _tc_mlp(part_ref, act_ref, W1_ref, b1_ref, W2_ref, b2T_ref, out_ref):
    pooled = part_ref[0] + part_ref[1] + act_ref[...]
    h1 = lax.dot_general(pooled, W1_ref[...],
                         (((1,), (1,)), ((), ()))) + b1_ref[...]
    h1 = jnp.maximum(h1, 0.0)
    yT = lax.dot_general(W2_ref[...], h1,
                         (((1,), (1,)), ((), ()))) + b2T_ref[...]
    m = jnp.max(yT, axis=0, keepdims=True)
    lse = m + jnp.log(jnp.sum(jnp.exp(yT - m), axis=0, keepdims=True))
    out_ref[...] = yT - lse


def kernel(params, acts, sign, global_idx, edge_index, batch):
    p = params
    gidx32 = global_idx.astype(jnp.int32)
    batch32 = batch.astype(jnp.int32)

    act_part = pl.pallas_call(
        _tc_acts,
        out_shape=jax.ShapeDtypeStruct((NG, C), jnp.float32),
    )(batch32.reshape(1, N), acts.T, p['pe_W'].T, p['pe_b'].reshape(1, C))

    partials = _sc_segsum()(p['emb'], gidx32, batch32)

    outT = pl.pallas_call(
        _tc_mlp,
        out_shape=jax.ShapeDtypeStruct((2, NG), jnp.float32),
    )(partials, act_part, p['fc1_W'], p['fc1_b'].reshape(1, 2 * C),
      p['fc2_W'], p['fc2_b'].reshape(2, 1))
    return outT.T


# speedup vs baseline: 1.0522x; 1.0062x over previous
import functools

import jax
import jax.numpy as jnp
from jax import lax
from jax.experimental import pallas as pl
from jax.experimental.pallas import tpu as pltpu
from jax.experimental.pallas import tpu_sc as plsc

N = 10000
C = 128
NG = 64
NW = 32
CHUNK = 104
NCH = 3
KPW = CHUNK * NCH
REM = N - NW * KPW
GPAD = 80


def _sc_body(emb_hbm, gidx_hbm, batch_hbm, out_hbm,
             gidx_v, bidx_v, rows_v, gidx_x, bidx_x, rows_x, zbuf_v, acc_sh,
             sem_sc, sem_x, sem_gi, sem_bi, *gsems):
    cid = lax.axis_index("c")
    sid = lax.axis_index("s")
    wid = sid * 2 + cid
    base = wid * KPW

    gload = pltpu.async_copy(gidx_hbm.at[pl.ds(base, KPW)], gidx_v, sem_gi)
    bload = pltpu.async_copy(batch_hbm.at[pl.ds(base, KPW)], bidx_v, sem_bi)
    for r in range(GPAD // 16):
        for k in range(C // 16):
            zbuf_v[r, pl.ds(k * 16, 16)] = jnp.zeros((16,), jnp.float32)
    pltpu.sync_copy(zbuf_v, acc_sh.at[pl.ds(sid * (GPAD // 16), GPAD // 16)])

    gload.wait()
    gathers = [
        pltpu.async_copy(emb_hbm.at[gidx_v.at[pl.ds(j * CHUNK, CHUNK)]],
                         rows_v.at[j], gsems[j])
        for j in range(NCH)
    ]
    bload.wait()
    plsc.subcore_barrier()

    @pl.when(wid == NW - 1)
    def _tail():
        tb = NW * KPW
        pltpu.sync_copy(gidx_hbm.at[pl.ds(tb, REM)], gidx_x)
        pltpu.sync_copy(batch_hbm.at[pl.ds(tb, REM)], bidx_x.at[0])
        pltpu.async_copy(emb_hbm.at[gidx_x], rows_x, sem_x).wait()
        pltpu.sync_copy(rows_x, acc_sh.at[bidx_x.at[0]], add=True)

    scatters = []
    for j in range(NCH):
        gathers[j].wait()
        scatters.append(pltpu.async_copy(
            rows_v.at[j], acc_sh.at[bidx_v.at[pl.ds(j * CHUNK, CHUNK)]],
            sem_sc, add=True))
    for d in scatters:
        d.wait()

    plsc.subcore_barrier()

    @pl.when(sid == 0)
    def _writeback():
        pltpu.sync_copy(acc_sh.at[pl.ds(0, NG)], out_hbm.at[cid])


@functools.cache
def _sc_segsum():
    mesh = plsc.VectorSubcoreMesh(core_axis_name="c", subcore_axis_name="s")
    return pl.kernel(
        _sc_body,
        out_type=jax.ShapeDtypeStruct((2, NG, C), jnp.float32),
        mesh=mesh,
        scratch_types=[
            pltpu.VMEM((KPW,), jnp.int32),
            pltpu.VMEM((KPW,), jnp.int32),
            pltpu.VMEM((NCH, CHUNK, C), jnp.float32),
            pltpu.VMEM((REM,), jnp.int32),
            pltpu.VMEM((1, REM), jnp.int32),
            pltpu.VMEM((REM, C), jnp.float32),
            pltpu.VMEM((GPAD // 16, C), jnp.float32),
            pltpu.VMEM_SHARED((GPAD, C), jnp.float32),
            pltpu.SemaphoreType.DMA,
            pltpu.SemaphoreType.DMA,
            pltpu.SemaphoreType.DMA,
            pltpu.SemaphoreType.DMA,
        ] + [pltpu.SemaphoreType.DMA] * NCH,
    )


def _tc_acts(batch_ref, actsT_ref, peWT_ref, peb_ref, out_ref):
    onehot = (batch_ref[...] == lax.broadcasted_iota(
        jnp.int32, (NG, N), 0)).astype(jnp.float32)
    segacts = lax.dot_general(onehot, actsT_ref[...],
                              (((1,), (1,)), ((), ())))
    counts = jnp.sum(onehot, axis=1, keepdims=True)
    out_ref[...] = lax.dot_general(segacts, peWT_ref[...],
                                   (((1,), (0,)), ((), ()))
                                   ) + counts * peb_ref[...]


def _tc_mlp(part_ref, act_ref, W1_ref, b1_ref, W2_ref, b2T_ref, out_ref):
    pooled = part_ref[0] + part_ref[1] + act_ref[...]
    h1 = lax.dot_general(pooled, W1_ref[...],
                         (((1,), (1,)), ((), ()))) + b1_ref[...]
    h1 = jnp.maximum(h1, 0.0)
    yT = lax.dot_general(W2_ref[...], h1,
                         (((1,), (1,)), ((), ()))) + b2T_ref[...]
    m = jnp.max(yT, axis=0, keepdims=True)
    lse = m + jnp.log(jnp.sum(jnp.exp(yT - m), axis=0, keepdims=True))
    out_ref[...] = yT - lse


def kernel(params, acts, sign, global_idx, edge_index, batch):
    p = params
    gidx32 = global_idx.astype(jnp.int32)
    batch32 = batch.astype(jnp.int32)

    act_part = pl.pallas_call(
        _tc_acts,
        out_shape=jax.ShapeDtypeStruct((NG, C), jnp.float32),
    )(batch32.reshape(1, N), acts.T, p['pe_W'].T, p['pe_b'].reshape(1, C))

    partials = _sc_segsum()(p['emb'], gidx32, batch32)

    outT = pl.pallas_call(
        _tc_mlp,
        out_shape=jax.ShapeDtypeStruct((2, NG), jnp.float32),
    )(partials, act_part, p['fc1_W'], p['fc1_b'].reshape(1, 2 * C),
      p['fc2_W'], p['fc2_b'].reshape(2, 1))
    return outT.T
